# Initial kernel scaffold; baseline (speedup 1.0000x reference)
import jax
import jax.numpy as jnp
from jax.experimental import pallas as pl


def kernel(x, edge_index, edge_attr, batch, W_embed, b_embed, conv0_W1, conv0_b1, conv0_W2, conv0_b2, conv1_W1, conv1_b1, conv1_W2, conv1_b2, conv2_W1, conv2_b1, conv2_W2, conv2_b2, W_mlp1, b_mlp1, W_mlp2, b_mlp2):
    raise NotImplementedError("write your pallas kernel here")


# speedup vs baseline: 2.1950x; 2.1950x over previous
import functools

import jax
import jax.numpy as jnp
from jax import lax
from jax.experimental import pallas as pl
from jax.experimental.pallas import tpu as pltpu
from jax.experimental.pallas import tpu_sc as plsc

F32 = jnp.float32
N = 10000
E = 320000
DF = 128
DE = 16
H = 128
G = 64

NC = 2
NS = 16
NW = NC * NS
CH = 128
NCHUNK = E // CH
ZCH = 80
NZCHUNK = N // ZCH

_SC_MESH = plsc.VectorSubcoreMesh(core_axis_name="c", subcore_axis_name="s")


@functools.partial(
    pl.kernel,
    mesh=_SC_MESH,
    out_type=[jax.ShapeDtypeStruct((E, H), F32),
              jax.ShapeDtypeStruct((E, H), F32)],
    scratch_types=[
        pltpu.VMEM((CH,), jnp.int32),
        pltpu.VMEM((CH,), jnp.int32),
        pltpu.VMEM((CH, H), F32),
        pltpu.VMEM((CH, H), F32),
        pltpu.SemaphoreType.DMA,
        pltpu.SemaphoreType.DMA,
    ],
)
def _sc_gather(hs_hbm, hd_hbm, src_hbm, dst_hbm, g1_hbm, g2_hbm,
               idx_s, idx_d, buf_a, buf_b, sem_a, sem_b):
    c = lax.axis_index("c")
    s = lax.axis_index("s")
    wid = s * NC + c
    nk = 78 + jnp.where(wid < NCHUNK - 78 * NW, 1, 0)

    def body(k, carry):
        base = (k * NW + wid) * CH
        pltpu.sync_copy(src_hbm.at[pl.ds(base, CH)], idx_s)
        pltpu.sync_copy(dst_hbm.at[pl.ds(base, CH)], idx_d)
        cp1 = pltpu.async_copy(hs_hbm.at[idx_s], buf_a, sem_a)
        cp2 = pltpu.async_copy(hd_hbm.at[idx_d], buf_b, sem_b)
        cp1.wait()
        cp2.wait()
        pltpu.sync_copy(buf_a, g1_hbm.at[pl.ds(base, CH)])
        pltpu.sync_copy(buf_b, g2_hbm.at[pl.ds(base, CH)])
        return carry

    lax.fori_loop(0, nk, body, 0)


@functools.partial(
    pl.kernel,
    mesh=_SC_MESH,
    out_type=jax.ShapeDtypeStruct((NC, N, H), F32),
    scratch_types=[
        pltpu.VMEM((CH,), jnp.int32),
        pltpu.VMEM((CH, H), F32),
        pltpu.VMEM_SHARED((N, H), F32),
    ],
)
def _sc_scatter(msg_hbm, dst_hbm, zrows_hbm, out_hbm, idx_d, buf, acc):
    c = lax.axis_index("c")
    s = lax.axis_index("s")
    nz = 7 + jnp.where(s < NZCHUNK - 7 * NS, 1, 0)

    def zbody(k, carry):
        row = (k * NS + s) * ZCH
        pltpu.sync_copy(zrows_hbm, acc.at[pl.ds(row, ZCH)])
        return carry

    lax.fori_loop(0, nz, zbody, 0)
    plsc.subcore_barrier()

    half = E // NC
    nch_core = half // CH
    nk = 78 + jnp.where(s < nch_core - 78 * NS, 1, 0)

    def body(k, carry):
        base = c * half + (k * NS + s) * CH
        pltpu.sync_copy(dst_hbm.at[pl.ds(base, CH)], idx_d)
        pltpu.sync_copy(msg_hbm.at[pl.ds(base, CH)], buf)
        pltpu.sync_copy(buf, acc.at[idx_d], add=True)
        return carry

    lax.fori_loop(0, nk, body, 0)
    plsc.subcore_barrier()

    def dbody(k, carry):
        row = (k * NS + s) * ZCH
        pltpu.sync_copy(acc.at[pl.ds(row, ZCH)],
                        out_hbm.at[c, pl.ds(row, ZCH)])
        return carry

    lax.fori_loop(0, nz, dbody, 0)


PN = 320


@functools.partial(
    pl.kernel,
    mesh=_SC_MESH,
    out_type=[jax.ShapeDtypeStruct((NW, G, H), F32),
              jax.ShapeDtypeStruct((NW, G, H), F32),
              jax.ShapeDtypeStruct((NW, G, 16), F32)],
    scratch_types=[
        pltpu.VMEM((PN, H), F32),
        pltpu.VMEM((G, H), F32),
        pltpu.VMEM((G, H), F32),
        pltpu.VMEM((G, 16), F32),
        pltpu.VMEM((PN,), jnp.int32),
    ],
)
def _sc_pool(h_hbm, batch_hbm, sum_out, max_out, cnt_out,
             hbuf, accs, accm, accc, bsm):
    c = lax.axis_index("c")
    s = lax.axis_index("s")
    wid = s * NC + c
    base = wid * PN
    nsub = jnp.where(wid < NW - 1, PN // 80, (N - (NW - 1) * PN) // 80)

    def cp(j, carry):
        pltpu.sync_copy(h_hbm.at[pl.ds(base + j * 80, 80)],
                        hbuf.at[pl.ds(j * 80, 80)])
        pltpu.sync_copy(batch_hbm.at[pl.ds(base + j * 80, 80)],
                        bsm.at[pl.ds(j * 80, 80)])
        return carry

    lax.fori_loop(0, nsub, cp, 0)

    zero16 = jnp.zeros((16,), F32)
    ninf16 = jnp.full((16,), -jnp.inf, F32)

    def zinit(r, carry):
        for j in range(H // 16):
            accs[r, pl.ds(j * 16, 16)] = zero16
            accm[r, pl.ds(j * 16, 16)] = ninf16
        accc[r, pl.ds(0, 16)] = zero16
        return carry

    lax.fori_loop(0, G, zinit, 0)

    one16 = jnp.full((16,), 1.0, F32)

    def grp(gi, carry):
        bv = bsm[pl.ds(gi * 16, 16)]
        for j in range(16):
            b = bv[j]
            n = gi * 16 + j
            for k in range(H // 16):
                hv = hbuf[n, pl.ds(k * 16, 16)]
                accs[b, pl.ds(k * 16, 16)] = accs[b, pl.ds(k * 16, 16)] + hv
                accm[b, pl.ds(k * 16, 16)] = jnp.maximum(
                    accm[b, pl.ds(k * 16, 16)], hv)
            accc[b, pl.ds(0, 16)] = accc[b, pl.ds(0, 16)] + one16
        return carry

    lax.fori_loop(0, nsub * 5, grp, 0)

    pltpu.sync_copy(accs, sum_out.at[wid])
    pltpu.sync_copy(accm, max_out.at[wid])
    pltpu.sync_copy(accc, cnt_out.at[wid])


BN = 1000
BE = 512


def _softplus(x):
    return jnp.log1p(jnp.exp(-jnp.abs(x))) + jnp.maximum(x, 0.0)


def _node0_body(x_ref, we_ref, be_ref, w1s_ref, w1d_ref, h_ref, hs_ref, hd_ref):
    h = jnp.dot(x_ref[...], we_ref[...], preferred_element_type=F32) + be_ref[...]
    h_ref[...] = h
    hs_ref[...] = jnp.dot(h, w1s_ref[...], preferred_element_type=F32)
    hd_ref[...] = jnp.dot(h, w1d_ref[...], preferred_element_type=F32)


def _node_mid_body(h_ref, a0_ref, a1_ref, w1s_ref, w1d_ref,
                   h_out, hs_ref, hd_ref):
    hn = _softplus(h_ref[...] + a0_ref[...] + a1_ref[...])
    h_out[...] = hn
    hs_ref[...] = jnp.dot(hn, w1s_ref[...], preferred_element_type=F32)
    hd_ref[...] = jnp.dot(hn, w1d_ref[...], preferred_element_type=F32)


def _node_fin_body(h_ref, a0_ref, a1_ref, h_out):
    h_out[...] = _softplus(h_ref[...] + a0_ref[...] + a1_ref[...])


def _edge_body(g1_ref, g2_ref, ea_ref, w1e_ref, b1_ref, w2_ref, b2_ref, msg_ref):
    pre = (g1_ref[...] + g2_ref[...] + b1_ref[...]
           + jnp.dot(ea_ref[...], w1e_ref[...], preferred_element_type=F32))
    e = jnp.maximum(pre, 0.0)
    t = jnp.dot(e, w2_ref[...], preferred_element_type=F32) + b2_ref[...]
    f = t[:, :H]
    cc = t[:, H:]
    msg_ref[...] = (1.0 / (1.0 + jnp.exp(-f))) * _softplus(cc)


def _combine_body(sum_ref, max_ref, cnt_ref, w1_ref, b1_ref, w2_ref, b2_ref,
                  out_ref):
    sm = jnp.sum(sum_ref[...], axis=0)
    mx = jnp.max(max_ref[...], axis=0)
    cnt = jnp.sum(cnt_ref[...], axis=0)[:, :1]
    mean = sm / jnp.maximum(cnt, 1.0)
    pooled = jnp.concatenate([mean, mx], axis=1)
    hid = jnp.maximum(
        jnp.dot(pooled, w1_ref[...], preferred_element_type=F32) + b1_ref[...],
        0.0)
    out_ref[...] = jnp.sum(hid * w2_ref[...], axis=1, keepdims=True) + b2_ref[...]


def _full(shape):
    return pl.BlockSpec(shape, lambda *args: (0,) * len(shape))


def _node0(x, we, be, w1s, w1d):
    return pl.pallas_call(
        _node0_body,
        grid=(N // BN,),
        in_specs=[pl.BlockSpec((BN, DF), lambda i: (i, 0)),
                  _full((DF, H)), _full((1, H)), _full((H, H)), _full((H, H))],
        out_specs=[pl.BlockSpec((BN, H), lambda i: (i, 0))] * 3,
        out_shape=[jax.ShapeDtypeStruct((N, H), F32)] * 3,
    )(x, we, be, w1s, w1d)


def _node_mid(h, a0, a1, w1s, w1d):
    return pl.pallas_call(
        _node_mid_body,
        grid=(N // BN,),
        in_specs=[pl.BlockSpec((BN, H), lambda i: (i, 0))] * 3
                 + [_full((H, H)), _full((H, H))],
        out_specs=[pl.BlockSpec((BN, H), lambda i: (i, 0))] * 3,
        out_shape=[jax.ShapeDtypeStruct((N, H), F32)] * 3,
    )(h, a0, a1, w1s, w1d)


def _node_fin(h, a0, a1):
    return pl.pallas_call(
        _node_fin_body,
        grid=(N // BN,),
        in_specs=[pl.BlockSpec((BN, H), lambda i: (i, 0))] * 3,
        out_specs=pl.BlockSpec((BN, H), lambda i: (i, 0)),
        out_shape=jax.ShapeDtypeStruct((N, H), F32),
    )(h, a0, a1)


def _edge_stage(g1, g2, ea, w1e, b1, w2, b2):
    return pl.pallas_call(
        _edge_body,
        grid=(E // BE,),
        in_specs=[pl.BlockSpec((BE, H), lambda i: (i, 0)),
                  pl.BlockSpec((BE, H), lambda i: (i, 0)),
                  pl.BlockSpec((BE, DE), lambda i: (i, 0)),
                  _full((DE, H)), _full((1, H)), _full((H, 2 * H)),
                  _full((1, 2 * H))],
        out_specs=pl.BlockSpec((BE, H), lambda i: (i, 0)),
        out_shape=jax.ShapeDtypeStruct((E, H), F32),
    )(g1, g2, ea, w1e, b1, w2, b2)


def _combine(sums, maxs, cnts, w1, b1, w2row, b2):
    return pl.pallas_call(
        _combine_body,
        in_specs=[_full((NW, G, H)), _full((NW, G, H)), _full((NW, G, 16)),
                  _full((2 * H, H)), _full((1, H)), _full((1, H)),
                  _full((1, 1))],
        out_specs=_full((G, 1)),
        out_shape=jax.ShapeDtypeStruct((G, 1), F32),
    )(sums, maxs, cnts, w1, b1, w2row, b2)


def kernel(x, edge_index, edge_attr, batch,
           W_embed, b_embed,
           conv0_W1, conv0_b1, conv0_W2, conv0_b2,
           conv1_W1, conv1_b1, conv1_W2, conv1_b2,
           conv2_W1, conv2_b1, conv2_W2, conv2_b2,
           W_mlp1, b_mlp1, W_mlp2, b_mlp2):
    src = edge_index[0]
    dst = edge_index[1]
    zrows = jnp.zeros((ZCH, H), F32)
    convs = [(conv0_W1, conv0_b1, conv0_W2, conv0_b2),
             (conv1_W1, conv1_b1, conv1_W2, conv1_b2),
             (conv2_W1, conv2_b1, conv2_W2, conv2_b2)]

    w1s0 = convs[0][0][:H]
    w1d0 = convs[0][0][H:2 * H]
    h, hs, hd = _node0(x, W_embed, b_embed.reshape(1, H), w1s0, w1d0)

    for l in range(3):
        W1, b1, W2, b2 = convs[l]
        g1, g2 = _sc_gather(hs, hd, src, dst)
        msg = _edge_stage(g1, g2, edge_attr, W1[2 * H:],
                          b1.reshape(1, H), W2, b2.reshape(1, 2 * H))
        aggp = _sc_scatter(msg, dst, zrows)
        if l < 2:
            w1s = convs[l + 1][0][:H]
            w1d = convs[l + 1][0][H:2 * H]
            h, hs, hd = _node_mid(h, aggp[0], aggp[1], w1s, w1d)
        else:
            h = _node_fin(h, aggp[0], aggp[1])

    sums, maxs, cnts = _sc_pool(h, batch)
    return _combine(sums, maxs, cnts, W_mlp1, b_mlp1.reshape(1, H),
                    W_mlp2.reshape(1, H), b_mlp2.reshape(1, 1))


# speedup vs baseline: 2.7679x; 1.2610x over previous
import functools

import jax
import jax.numpy as jnp
from jax import lax
from jax.experimental import pallas as pl
from jax.experimental.pallas import tpu as pltpu
from jax.experimental.pallas import tpu_sc as plsc

F32 = jnp.float32
N = 10000
E = 320000
DF = 128
DE = 16
H = 128
G = 64

NC = 2
NS = 16
NW = NC * NS
CH = 128
NCHUNK = E // CH
ZCH = 80
NZCHUNK = N // ZCH

_SC_MESH = plsc.VectorSubcoreMesh(core_axis_name="c", subcore_axis_name="s")


@functools.partial(
    pl.kernel,
    mesh=_SC_MESH,
    out_type=jax.ShapeDtypeStruct((E, H), F32),
    scratch_types=[
        pltpu.VMEM((CH,), jnp.int32), pltpu.VMEM((CH,), jnp.int32),
        pltpu.VMEM((CH,), jnp.int32), pltpu.VMEM((CH,), jnp.int32),
        pltpu.VMEM((CH, H), F32), pltpu.VMEM((CH, H), F32),
        pltpu.VMEM((CH, H), F32), pltpu.VMEM((CH, H), F32),
        pltpu.SemaphoreType.DMA, pltpu.SemaphoreType.DMA,
        pltpu.SemaphoreType.DMA, pltpu.SemaphoreType.DMA,
    ],
)
def _sc_gather(hs_hbm, hd_hbm, src_hbm, dst_hbm, g_hbm,
               is0, id0, is1, id1, a0, b0, a1, b1,
               sa0, sb0, sa1, sb1):
    c = lax.axis_index("c")
    s = lax.axis_index("s")
    wid = s * NC + c
    idx_s = (is0, is1)
    idx_d = (id0, id1)
    buf_a = (a0, a1)
    buf_b = (b0, b1)
    sem_a = (sa0, sa1)
    sem_b = (sb0, sb1)

    def issue(k, slot):
        base = (k * NW + wid) * CH
        pltpu.sync_copy(src_hbm.at[pl.ds(base, CH)], idx_s[slot])
        pltpu.sync_copy(dst_hbm.at[pl.ds(base, CH)], idx_d[slot])
        pltpu.async_copy(hs_hbm.at[idx_s[slot]], buf_a[slot], sem_a[slot])
        pltpu.async_copy(hd_hbm.at[idx_d[slot]], buf_b[slot], sem_b[slot])

    def finish(k, slot):
        base = (k * NW + wid) * CH
        pltpu.make_async_copy(hs_hbm.at[idx_s[slot]], buf_a[slot],
                              sem_a[slot]).wait()
        pltpu.make_async_copy(hd_hbm.at[idx_d[slot]], buf_b[slot],
                              sem_b[slot]).wait()
        ba, bb = buf_a[slot], buf_b[slot]

        def addrow(r, carry):
            for j in range(H // 16):
                ba[r, pl.ds(j * 16, 16)] = (ba[r, pl.ds(j * 16, 16)]
                                            + bb[r, pl.ds(j * 16, 16)])
            return carry

        lax.fori_loop(0, CH, addrow, 0)
        pltpu.sync_copy(ba, g_hbm.at[pl.ds(base, CH)])

    issue(0, 0)

    def pair(gi, carry):
        k0 = gi * 2
        issue(k0 + 1, 1)
        finish(k0, 0)

        @pl.when(gi < 38)
        def _():
            issue(k0 + 2, 0)

        finish(k0 + 1, 1)
        return carry

    lax.fori_loop(0, 39, pair, 0)

    @pl.when(wid < NCHUNK - 78 * NW)
    def _():
        base = (NCHUNK - 4 + wid) * CH
        pltpu.sync_copy(src_hbm.at[pl.ds(base, CH)], is0)
        pltpu.sync_copy(dst_hbm.at[pl.ds(base, CH)], id0)
        pltpu.async_copy(hs_hbm.at[is0], a0, sa0).wait()
        pltpu.async_copy(hd_hbm.at[id0], b0, sb0).wait()

        def addrow(r, carry):
            for j in range(H // 16):
                a0[r, pl.ds(j * 16, 16)] = (a0[r, pl.ds(j * 16, 16)]
                                            + b0[r, pl.ds(j * 16, 16)])
            return carry

        lax.fori_loop(0, CH, addrow, 0)
        pltpu.sync_copy(a0, g_hbm.at[pl.ds(base, CH)])


@functools.partial(
    pl.kernel,
    mesh=_SC_MESH,
    out_type=jax.ShapeDtypeStruct((NC, N, H), F32),
    scratch_types=[
        pltpu.VMEM((CH,), jnp.int32), pltpu.VMEM((CH,), jnp.int32),
        pltpu.VMEM((CH, H), F32), pltpu.VMEM((CH, H), F32),
        pltpu.VMEM_SHARED((N, H), F32),
        pltpu.SemaphoreType.DMA, pltpu.SemaphoreType.DMA,
        pltpu.SemaphoreType.DMA, pltpu.SemaphoreType.DMA,
    ],
)
def _sc_scatter(msg_hbm, dst_hbm, zrows_hbm, out_hbm,
                id0, id1, m0, m1, acc, si0, si1, sm0, sm1):
    c = lax.axis_index("c")
    s = lax.axis_index("s")
    nz = 7 + jnp.where(s < NZCHUNK - 7 * NS, 1, 0)

    def zbody(k, carry):
        row = (k * NS + s) * ZCH
        pltpu.sync_copy(zrows_hbm, acc.at[pl.ds(row, ZCH)])
        return carry

    lax.fori_loop(0, nz, zbody, 0)
    plsc.subcore_barrier()

    half = E // NC
    nch_core = half // CH
    idx_d = (id0, id1)
    mbuf = (m0, m1)
    sem_i = (si0, si1)
    sem_m = (sm0, sm1)

    def issue(k, slot):
        base = c * half + (k * NS + s) * CH
        pltpu.async_copy(dst_hbm.at[pl.ds(base, CH)], idx_d[slot], sem_i[slot])
        pltpu.async_copy(msg_hbm.at[pl.ds(base, CH)], mbuf[slot], sem_m[slot])

    def finish(k, slot):
        base = c * half + (k * NS + s) * CH
        pltpu.make_async_copy(dst_hbm.at[pl.ds(base, CH)], idx_d[slot],
                              sem_i[slot]).wait()
        pltpu.make_async_copy(msg_hbm.at[pl.ds(base, CH)], mbuf[slot],
                              sem_m[slot]).wait()
        pltpu.sync_copy(mbuf[slot], acc.at[idx_d[slot]], add=True)

    issue(0, 0)

    def pair(gi, carry):
        k0 = gi * 2
        issue(k0 + 1, 1)
        finish(k0, 0)

        @pl.when(gi < 38)
        def _():
            issue(k0 + 2, 0)

        finish(k0 + 1, 1)
        return carry

    lax.fori_loop(0, 39, pair, 0)

    @pl.when(s < nch_core - 78 * NS)
    def _():
        base = c * half + (nch_core - 2 + s) * CH
        pltpu.sync_copy(dst_hbm.at[pl.ds(base, CH)], id0)
        pltpu.sync_copy(msg_hbm.at[pl.ds(base, CH)], m0)
        pltpu.sync_copy(m0, acc.at[id0], add=True)

    plsc.subcore_barrier()

    def dbody(k, carry):
        row = (k * NS + s) * ZCH
        pltpu.sync_copy(acc.at[pl.ds(row, ZCH)],
                        out_hbm.at[c, pl.ds(row, ZCH)])
        return carry

    lax.fori_loop(0, nz, dbody, 0)


PN = 320


@functools.partial(
    pl.kernel,
    mesh=_SC_MESH,
    out_type=[jax.ShapeDtypeStruct((NW, G, H), F32),
              jax.ShapeDtypeStruct((NW, G, H), F32),
              jax.ShapeDtypeStruct((NW, G, 16), F32)],
    scratch_types=[
        pltpu.VMEM((PN, H), F32),
        pltpu.VMEM((G, H), F32),
        pltpu.VMEM((G, H), F32),
        pltpu.VMEM((G, 16), F32),
        pltpu.VMEM((PN,), jnp.int32),
    ],
)
def _sc_pool(h_hbm, batch_hbm, sum_out, max_out, cnt_out,
             hbuf, accs, accm, accc, bsm):
    c = lax.axis_index("c")
    s = lax.axis_index("s")
    wid = s * NC + c
    base = wid * PN
    nsub = jnp.where(wid < NW - 1, PN // 80, (N - (NW - 1) * PN) // 80)

    def cp(j, carry):
        pltpu.sync_copy(h_hbm.at[pl.ds(base + j * 80, 80)],
                        hbuf.at[pl.ds(j * 80, 80)])
        pltpu.sync_copy(batch_hbm.at[pl.ds(base + j * 80, 80)],
                        bsm.at[pl.ds(j * 80, 80)])
        return carry

    lax.fori_loop(0, nsub, cp, 0)

    zero16 = jnp.zeros((16,), F32)
    ninf16 = jnp.full((16,), -jnp.inf, F32)

    def zinit(r, carry):
        for j in range(H // 16):
            accs[r, pl.ds(j * 16, 16)] = zero16
            accm[r, pl.ds(j * 16, 16)] = ninf16
        accc[r, pl.ds(0, 16)] = zero16
        return carry

    lax.fori_loop(0, G, zinit, 0)

    one16 = jnp.full((16,), 1.0, F32)

    def grp(gi, carry):
        bv = bsm[pl.ds(gi * 16, 16)]
        for j in range(16):
            b = bv[j]
            n = gi * 16 + j
            for k in range(H // 16):
                hv = hbuf[n, pl.ds(k * 16, 16)]
                accs[b, pl.ds(k * 16, 16)] = accs[b, pl.ds(k * 16, 16)] + hv
                accm[b, pl.ds(k * 16, 16)] = jnp.maximum(
                    accm[b, pl.ds(k * 16, 16)], hv)
            accc[b, pl.ds(0, 16)] = accc[b, pl.ds(0, 16)] + one16
        return carry

    lax.fori_loop(0, nsub * 5, grp, 0)

    pltpu.sync_copy(accs, sum_out.at[wid])
    pltpu.sync_copy(accm, max_out.at[wid])
    pltpu.sync_copy(accc, cnt_out.at[wid])


BN = 1000
BE = 512


def _softplus(x):
    return jnp.log1p(jnp.exp(-jnp.abs(x))) + jnp.maximum(x, 0.0)


def _node0_body(x_ref, we_ref, be_ref, w1s_ref, w1d_ref, h_ref, hs_ref, hd_ref):
    h = jnp.dot(x_ref[...], we_ref[...], preferred_element_type=F32) + be_ref[...]
    h_ref[...] = h
    hs_ref[...] = jnp.dot(h, w1s_ref[...], preferred_element_type=F32)
    hd_ref[...] = jnp.dot(h, w1d_ref[...], preferred_element_type=F32)


def _node_mid_body(h_ref, a0_ref, a1_ref, w1s_ref, w1d_ref,
                   h_out, hs_ref, hd_ref):
    hn = _softplus(h_ref[...] + a0_ref[...] + a1_ref[...])
    h_out[...] = hn
    hs_ref[...] = jnp.dot(hn, w1s_ref[...], preferred_element_type=F32)
    hd_ref[...] = jnp.dot(hn, w1d_ref[...], preferred_element_type=F32)


def _node_fin_body(h_ref, a0_ref, a1_ref, h_out):
    h_out[...] = _softplus(h_ref[...] + a0_ref[...] + a1_ref[...])


def _edge_body(g_ref, ea_ref, w1e_ref, b1_ref, w2_ref, b2_ref, msg_ref):
    pre = (g_ref[...] + b1_ref[...]
           + jnp.dot(ea_ref[...], w1e_ref[...], preferred_element_type=F32))
    e = jnp.maximum(pre, 0.0)
    t = jnp.dot(e, w2_ref[...], preferred_element_type=F32) + b2_ref[...]
    f = t[:, :H]
    cc = t[:, H:]
    msg_ref[...] = (1.0 / (1.0 + jnp.exp(-f))) * _softplus(cc)


def _combine_body(sum_ref, max_ref, cnt_ref, w1_ref, b1_ref, w2_ref, b2_ref,
                  out_ref):
    sm = jnp.sum(sum_ref[...], axis=0)
    mx = jnp.max(max_ref[...], axis=0)
    cnt = jnp.sum(cnt_ref[...], axis=0)[:, :1]
    mean = sm / jnp.maximum(cnt, 1.0)
    pooled = jnp.concatenate([mean, mx], axis=1)
    hid = jnp.maximum(
        jnp.dot(pooled, w1_ref[...], preferred_element_type=F32) + b1_ref[...],
        0.0)
    out_ref[...] = jnp.sum(hid * w2_ref[...], axis=1, keepdims=True) + b2_ref[...]


def _full(shape):
    return pl.BlockSpec(shape, lambda *args: (0,) * len(shape))


def _node0(x, we, be, w1s, w1d):
    return pl.pallas_call(
        _node0_body,
        grid=(N // BN,),
        in_specs=[pl.BlockSpec((BN, DF), lambda i: (i, 0)),
                  _full((DF, H)), _full((1, H)), _full((H, H)), _full((H, H))],
        out_specs=[pl.BlockSpec((BN, H), lambda i: (i, 0))] * 3,
        out_shape=[jax.ShapeDtypeStruct((N, H), F32)] * 3,
    )(x, we, be, w1s, w1d)


def _node_mid(h, a0, a1, w1s, w1d):
    return pl.pallas_call(
        _node_mid_body,
        grid=(N // BN,),
        in_specs=[pl.BlockSpec((BN, H), lambda i: (i, 0))] * 3
                 + [_full((H, H)), _full((H, H))],
        out_specs=[pl.BlockSpec((BN, H), lambda i: (i, 0))] * 3,
        out_shape=[jax.ShapeDtypeStruct((N, H), F32)] * 3,
    )(h, a0, a1, w1s, w1d)


def _node_fin(h, a0, a1):
    return pl.pallas_call(
        _node_fin_body,
        grid=(N // BN,),
        in_specs=[pl.BlockSpec((BN, H), lambda i: (i, 0))] * 3,
        out_specs=pl.BlockSpec((BN, H), lambda i: (i, 0)),
        out_shape=jax.ShapeDtypeStruct((N, H), F32),
    )(h, a0, a1)


def _edge_stage(g, ea, w1e, b1, w2, b2):
    return pl.pallas_call(
        _edge_body,
        grid=(E // BE,),
        in_specs=[pl.BlockSpec((BE, H), lambda i: (i, 0)),
                  pl.BlockSpec((BE, DE), lambda i: (i, 0)),
                  _full((DE, H)), _full((1, H)), _full((H, 2 * H)),
                  _full((1, 2 * H))],
        out_specs=pl.BlockSpec((BE, H), lambda i: (i, 0)),
        out_shape=jax.ShapeDtypeStruct((E, H), F32),
    )(g, ea, w1e, b1, w2, b2)


def _combine(sums, maxs, cnts, w1, b1, w2row, b2):
    return pl.pallas_call(
        _combine_body,
        in_specs=[_full((NW, G, H)), _full((NW, G, H)), _full((NW, G, 16)),
                  _full((2 * H, H)), _full((1, H)), _full((1, H)),
                  _full((1, 1))],
        out_specs=_full((G, 1)),
        out_shape=jax.ShapeDtypeStruct((G, 1), F32),
    )(sums, maxs, cnts, w1, b1, w2row, b2)


def kernel(x, edge_index, edge_attr, batch,
           W_embed, b_embed,
           conv0_W1, conv0_b1, conv0_W2, conv0_b2,
           conv1_W1, conv1_b1, conv1_W2, conv1_b2,
           conv2_W1, conv2_b1, conv2_W2, conv2_b2,
           W_mlp1, b_mlp1, W_mlp2, b_mlp2):
    src = edge_index[0]
    dst = edge_index[1]
    zrows = jnp.zeros((ZCH, H), F32)
    convs = [(conv0_W1, conv0_b1, conv0_W2, conv0_b2),
             (conv1_W1, conv1_b1, conv1_W2, conv1_b2),
             (conv2_W1, conv2_b1, conv2_W2, conv2_b2)]

    w1s0 = convs[0][0][:H]
    w1d0 = convs[0][0][H:2 * H]
    h, hs, hd = _node0(x, W_embed, b_embed.reshape(1, H), w1s0, w1d0)

    for l in range(3):
        W1, b1, W2, b2 = convs[l]
        g = _sc_gather(hs, hd, src, dst)
        msg = _edge_stage(g, edge_attr, W1[2 * H:],
                          b1.reshape(1, H), W2, b2.reshape(1, 2 * H))
        aggp = _sc_scatter(msg, dst, zrows)
        if l < 2:
            w1s = convs[l + 1][0][:H]
            w1d = convs[l + 1][0][H:2 * H]
            h, hs, hd = _node_mid(h, aggp[0], aggp[1], w1s, w1d)
        else:
            h = _node_fin(h, aggp[0], aggp[1])

    sums, maxs, cnts = _sc_pool(h, batch)
    return _combine(sums, maxs, cnts, W_mlp1, b_mlp1.reshape(1, H),
                    W_mlp2.reshape(1, H), b_mlp2.reshape(1, 1))


# speedup vs baseline: 3.3871x; 1.2237x over previous
import functools

import jax
import jax.numpy as jnp
from jax import lax
from jax.experimental import pallas as pl
from jax.experimental.pallas import tpu as pltpu
from jax.experimental.pallas import tpu_sc as plsc

F32 = jnp.float32
N = 10000
E = 320000
EH = E // 2
DF = 128
DE = 16
H = 128
G = 64

NC = 2
NS = 16
NW = NC * NS
CH = 128
ZCH = 80
NZCHUNK = N // ZCH

_SC_MESH = plsc.VectorSubcoreMesh(core_axis_name="c", subcore_axis_name="s")


def _make_gather(ne):
    nch = ne // CH
    per = nch // NW
    lft = nch - per * NW
    npair = per // 2

    @functools.partial(
        pl.kernel,
        mesh=_SC_MESH,
        out_type=jax.ShapeDtypeStruct((ne, H), F32),
        scratch_types=[
            pltpu.VMEM((CH,), jnp.int32), pltpu.VMEM((CH,), jnp.int32),
            pltpu.VMEM((CH,), jnp.int32), pltpu.VMEM((CH,), jnp.int32),
            pltpu.VMEM((CH, H), F32), pltpu.VMEM((CH, H), F32),
            pltpu.VMEM((CH, H), F32), pltpu.VMEM((CH, H), F32),
            pltpu.SemaphoreType.DMA, pltpu.SemaphoreType.DMA,
            pltpu.SemaphoreType.DMA, pltpu.SemaphoreType.DMA,
        ],
    )
    def gather(hs_hbm, hd_hbm, src_hbm, dst_hbm, g_hbm,
               is0, id0, is1, id1, a0, b0, a1, b1,
               sa0, sb0, sa1, sb1):
        c = lax.axis_index("c")
        s = lax.axis_index("s")
        wid = s * NC + c
        idx_s = (is0, is1)
        idx_d = (id0, id1)
        buf_a = (a0, a1)
        buf_b = (b0, b1)
        sem_a = (sa0, sa1)
        sem_b = (sb0, sb1)

        def _add(ba, bb):
            def addrow(r, carry):
                for j in range(H // 16):
                    ba[r, pl.ds(j * 16, 16)] = (ba[r, pl.ds(j * 16, 16)]
                                                + bb[r, pl.ds(j * 16, 16)])
                return carry

            lax.fori_loop(0, CH, addrow, 0)

        def issue(k, slot):
            base = (k * NW + wid) * CH
            pltpu.sync_copy(src_hbm.at[pl.ds(base, CH)], idx_s[slot])
            pltpu.sync_copy(dst_hbm.at[pl.ds(base, CH)], idx_d[slot])
            pltpu.async_copy(hs_hbm.at[idx_s[slot]], buf_a[slot], sem_a[slot])
            pltpu.async_copy(hd_hbm.at[idx_d[slot]], buf_b[slot], sem_b[slot])

        def finish(k, slot):
            base = (k * NW + wid) * CH
            pltpu.make_async_copy(hs_hbm.at[idx_s[slot]], buf_a[slot],
                                  sem_a[slot]).wait()
            pltpu.make_async_copy(hd_hbm.at[idx_d[slot]], buf_b[slot],
                                  sem_b[slot]).wait()
            _add(buf_a[slot], buf_b[slot])
            pltpu.sync_copy(buf_a[slot], g_hbm.at[pl.ds(base, CH)])

        issue(0, 0)

        def pair(gi, carry):
            k0 = gi * 2
            issue(k0 + 1, 1)
            finish(k0, 0)

            @pl.when(k0 + 2 < per)
            def _():
                issue(k0 + 2, 0)

            finish(k0 + 1, 1)
            return carry

        lax.fori_loop(0, npair, pair, 0)

        if per % 2 == 1:
            finish(per - 1, 0)

        if lft:
            @pl.when(wid < lft)
            def _():
                base = (per * NW + wid) * CH
                pltpu.sync_copy(src_hbm.at[pl.ds(base, CH)], is1)
                pltpu.sync_copy(dst_hbm.at[pl.ds(base, CH)], id1)
                pltpu.async_copy(hs_hbm.at[is1], a1, sa1).wait()
                pltpu.async_copy(hd_hbm.at[id1], b1, sb1).wait()
                _add(a1, b1)
                pltpu.sync_copy(a1, g_hbm.at[pl.ds(base, CH)])

    return gather


def _make_scatter(ne):
    half = ne // NC
    nchc = half // CH
    per = nchc // NS
    lft = nchc - per * NS
    npair = per // 2

    @functools.partial(
        pl.kernel,
        mesh=_SC_MESH,
        out_type=jax.ShapeDtypeStruct((NC, N, H), F32),
        scratch_types=[
            pltpu.VMEM((CH,), jnp.int32), pltpu.VMEM((CH,), jnp.int32),
            pltpu.VMEM((CH, H), F32), pltpu.VMEM((CH, H), F32),
            pltpu.VMEM_SHARED((N, H), F32),
            pltpu.SemaphoreType.DMA, pltpu.SemaphoreType.DMA,
            pltpu.SemaphoreType.DMA, pltpu.SemaphoreType.DMA,
        ],
    )
    def scatter(msg_hbm, dst_hbm, zrows_hbm, out_hbm,
                id0, id1, m0, m1, acc, si0, si1, sm0, sm1):
        c = lax.axis_index("c")
        s = lax.axis_index("s")
        nz = 7 + jnp.where(s < NZCHUNK - 7 * NS, 1, 0)

        def zbody(k, carry):
            row = (k * NS + s) * ZCH
            pltpu.sync_copy(zrows_hbm, acc.at[pl.ds(row, ZCH)])
            return carry

        lax.fori_loop(0, nz, zbody, 0)
        plsc.subcore_barrier()

        idx_d = (id0, id1)
        mbuf = (m0, m1)
        sem_i = (si0, si1)
        sem_m = (sm0, sm1)

        def issue(k, slot):
            base = c * half + (k * NS + s) * CH
            pltpu.async_copy(dst_hbm.at[pl.ds(base, CH)], idx_d[slot],
                             sem_i[slot])
            pltpu.async_copy(msg_hbm.at[pl.ds(base, CH)], mbuf[slot],
                             sem_m[slot])

        def finish(k, slot):
            base = c * half + (k * NS + s) * CH
            pltpu.make_async_copy(dst_hbm.at[pl.ds(base, CH)], idx_d[slot],
                                  sem_i[slot]).wait()
            pltpu.make_async_copy(msg_hbm.at[pl.ds(base, CH)], mbuf[slot],
                                  sem_m[slot]).wait()
            pltpu.sync_copy(mbuf[slot], acc.at[idx_d[slot]], add=True)

        issue(0, 0)

        def pair(gi, carry):
            k0 = gi * 2
            issue(k0 + 1, 1)
            finish(k0, 0)

            @pl.when(k0 + 2 < per)
            def _():
                issue(k0 + 2, 0)

            finish(k0 + 1, 1)
            return carry

        lax.fori_loop(0, npair, pair, 0)

        if per % 2 == 1:
            finish(per - 1, 0)

        if lft:
            @pl.when(s < lft)
            def _():
                base = c * half + (per * NS + s) * CH
                pltpu.sync_copy(dst_hbm.at[pl.ds(base, CH)], id1)
                pltpu.sync_copy(msg_hbm.at[pl.ds(base, CH)], m1)
                pltpu.sync_copy(m1, acc.at[id1], add=True)

        plsc.subcore_barrier()

        def dbody(k, carry):
            row = (k * NS + s) * ZCH
            pltpu.sync_copy(acc.at[pl.ds(row, ZCH)],
                            out_hbm.at[c, pl.ds(row, ZCH)])
            return carry

        lax.fori_loop(0, nz, dbody, 0)

    return scatter


_sc_gather_h = _make_gather(EH)
_sc_scatter_h = _make_scatter(EH)


PN = 320


@functools.partial(
    pl.kernel,
    mesh=_SC_MESH,
    out_type=[jax.ShapeDtypeStruct((NW, G, H), F32),
              jax.ShapeDtypeStruct((NW, G, H), F32),
              jax.ShapeDtypeStruct((NW, G, 16), F32)],
    scratch_types=[
        pltpu.VMEM((PN, H), F32),
        pltpu.VMEM((G, H), F32),
        pltpu.VMEM((G, H), F32),
        pltpu.VMEM((G, 16), F32),
        pltpu.VMEM((PN,), jnp.int32),
    ],
)
def _sc_pool(h_hbm, batch_hbm, sum_out, max_out, cnt_out,
             hbuf, accs, accm, accc, bsm):
    c = lax.axis_index("c")
    s = lax.axis_index("s")
    wid = s * NC + c
    base = wid * PN
    nsub = jnp.where(wid < NW - 1, PN // 80, (N - (NW - 1) * PN) // 80)

    def cp(j, carry):
        pltpu.sync_copy(h_hbm.at[pl.ds(base + j * 80, 80)],
                        hbuf.at[pl.ds(j * 80, 80)])
        pltpu.sync_copy(batch_hbm.at[pl.ds(base + j * 80, 80)],
                        bsm.at[pl.ds(j * 80, 80)])
        return carry

    lax.fori_loop(0, nsub, cp, 0)

    zero16 = jnp.zeros((16,), F32)
    ninf16 = jnp.full((16,), -jnp.inf, F32)

    def zinit(r, carry):
        for j in range(H // 16):
            accs[r, pl.ds(j * 16, 16)] = zero16
            accm[r, pl.ds(j * 16, 16)] = ninf16
        accc[r, pl.ds(0, 16)] = zero16
        return carry

    lax.fori_loop(0, G, zinit, 0)

    one16 = jnp.full((16,), 1.0, F32)

    def grp(gi, carry):
        bv = bsm[pl.ds(gi * 16, 16)]
        for j in range(16):
            b = bv[j]
            n = gi * 16 + j
            for k in range(H // 16):
                hv = hbuf[n, pl.ds(k * 16, 16)]
                accs[b, pl.ds(k * 16, 16)] = accs[b, pl.ds(k * 16, 16)] + hv
                accm[b, pl.ds(k * 16, 16)] = jnp.maximum(
                    accm[b, pl.ds(k * 16, 16)], hv)
            accc[b, pl.ds(0, 16)] = accc[b, pl.ds(0, 16)] + one16
        return carry

    lax.fori_loop(0, nsub * 5, grp, 0)

    pltpu.sync_copy(accs, sum_out.at[wid])
    pltpu.sync_copy(accm, max_out.at[wid])
    pltpu.sync_copy(accc, cnt_out.at[wid])


BN = 1000
BE = 640


def _softplus(x):
    return jnp.log1p(jnp.exp(-jnp.abs(x))) + jnp.maximum(x, 0.0)


def _node0_body(x_ref, we_ref, be_ref, w1s_ref, w1d_ref, h_ref, hs_ref, hd_ref):
    h = jnp.dot(x_ref[...], we_ref[...], preferred_element_type=F32) + be_ref[...]
    h_ref[...] = h
    hs_ref[...] = jnp.dot(h, w1s_ref[...], preferred_element_type=F32)
    hd_ref[...] = jnp.dot(h, w1d_ref[...], preferred_element_type=F32)


def _node_mid_body(h_ref, aa_ref, ab_ref, w1s_ref, w1d_ref,
                   h_out, hs_ref, hd_ref):
    hn = _softplus(h_ref[...] + aa_ref[0] + aa_ref[1]
                   + ab_ref[0] + ab_ref[1])
    h_out[...] = hn
    hs_ref[...] = jnp.dot(hn, w1s_ref[...], preferred_element_type=F32)
    hd_ref[...] = jnp.dot(hn, w1d_ref[...], preferred_element_type=F32)


def _node_fin_body(h_ref, aa_ref, ab_ref, h_out):
    h_out[...] = _softplus(h_ref[...] + aa_ref[0] + aa_ref[1]
                           + ab_ref[0] + ab_ref[1])


def _edge_body(g_ref, ea_ref, w1e_ref, b1_ref, w2_ref, b2_ref, msg_ref):
    pre = (g_ref[...] + b1_ref[...]
           + jnp.dot(ea_ref[...], w1e_ref[...], preferred_element_type=F32))
    e = jnp.maximum(pre, 0.0)
    t = jnp.dot(e, w2_ref[...], preferred_element_type=F32) + b2_ref[...]
    f = t[:, :H]
    cc = t[:, H:]
    msg_ref[...] = (1.0 / (1.0 + jnp.exp(-f))) * _softplus(cc)


def _combine_body(sum_ref, max_ref, cnt_ref, w1_ref, b1_ref, w2_ref, b2_ref,
                  out_ref):
    sm = jnp.sum(sum_ref[...], axis=0)
    mx = jnp.max(max_ref[...], axis=0)
    cnt = jnp.sum(cnt_ref[...], axis=0)[:, :1]
    mean = sm / jnp.maximum(cnt, 1.0)
    pooled = jnp.concatenate([mean, mx], axis=1)
    hid = jnp.maximum(
        jnp.dot(pooled, w1_ref[...], preferred_element_type=F32) + b1_ref[...],
        0.0)
    out_ref[...] = jnp.sum(hid * w2_ref[...], axis=1, keepdims=True) + b2_ref[...]


def _full(shape):
    return pl.BlockSpec(shape, lambda *args: (0,) * len(shape))


_AGG_SPEC = pl.BlockSpec((NC, BN, H), lambda i: (0, i, 0))


def _node0(x, we, be, w1s, w1d):
    return pl.pallas_call(
        _node0_body,
        grid=(N // BN,),
        in_specs=[pl.BlockSpec((BN, DF), lambda i: (i, 0)),
                  _full((DF, H)), _full((1, H)), _full((H, H)), _full((H, H))],
        out_specs=[pl.BlockSpec((BN, H), lambda i: (i, 0))] * 3,
        out_shape=[jax.ShapeDtypeStruct((N, H), F32)] * 3,
    )(x, we, be, w1s, w1d)


def _node_mid(h, agg_a, agg_b, w1s, w1d):
    return pl.pallas_call(
        _node_mid_body,
        grid=(N // BN,),
        in_specs=[pl.BlockSpec((BN, H), lambda i: (i, 0)),
                  _AGG_SPEC, _AGG_SPEC,
                  _full((H, H)), _full((H, H))],
        out_specs=[pl.BlockSpec((BN, H), lambda i: (i, 0))] * 3,
        out_shape=[jax.ShapeDtypeStruct((N, H), F32)] * 3,
    )(h, agg_a, agg_b, w1s, w1d)


def _node_fin(h, agg_a, agg_b):
    return pl.pallas_call(
        _node_fin_body,
        grid=(N // BN,),
        in_specs=[pl.BlockSpec((BN, H), lambda i: (i, 0)),
                  _AGG_SPEC, _AGG_SPEC],
        out_specs=pl.BlockSpec((BN, H), lambda i: (i, 0)),
        out_shape=jax.ShapeDtypeStruct((N, H), F32),
    )(h, agg_a, agg_b)


def _edge_stage(g, ea_full, half_idx, w1e, b1, w2, b2):
    nblk = EH // BE
    off = half_idx * nblk

    return pl.pallas_call(
        _edge_body,
        grid=(nblk,),
        in_specs=[pl.BlockSpec((BE, H), lambda i: (i, 0)),
                  pl.BlockSpec((BE, DE), lambda i: (i + off, 0)),
                  _full((DE, H)), _full((1, H)), _full((H, 2 * H)),
                  _full((1, 2 * H))],
        out_specs=pl.BlockSpec((BE, H), lambda i: (i, 0)),
        out_shape=jax.ShapeDtypeStruct((EH, H), F32),
    )(g, ea_full, w1e, b1, w2, b2)


def _combine(sums, maxs, cnts, w1, b1, w2row, b2):
    return pl.pallas_call(
        _combine_body,
        in_specs=[_full((NW, G, H)), _full((NW, G, H)), _full((NW, G, 16)),
                  _full((2 * H, H)), _full((1, H)), _full((1, H)),
                  _full((1, 1))],
        out_specs=_full((G, 1)),
        out_shape=jax.ShapeDtypeStruct((G, 1), F32),
    )(sums, maxs, cnts, w1, b1, w2row, b2)


def kernel(x, edge_index, edge_attr, batch,
           W_embed, b_embed,
           conv0_W1, conv0_b1, conv0_W2, conv0_b2,
           conv1_W1, conv1_b1, conv1_W2, conv1_b2,
           conv2_W1, conv2_b1, conv2_W2, conv2_b2,
           W_mlp1, b_mlp1, W_mlp2, b_mlp2):
    src = edge_index[0]
    dst = edge_index[1]
    src_h = (src[:EH], src[EH:])
    dst_h = (dst[:EH], dst[EH:])
    zrows = jnp.zeros((ZCH, H), F32)
    convs = [(conv0_W1, conv0_b1, conv0_W2, conv0_b2),
             (conv1_W1, conv1_b1, conv1_W2, conv1_b2),
             (conv2_W1, conv2_b1, conv2_W2, conv2_b2)]

    w1s0 = convs[0][0][:H]
    w1d0 = convs[0][0][H:2 * H]
    h, hs, hd = _node0(x, W_embed, b_embed.reshape(1, H), w1s0, w1d0)

    for l in range(3):
        W1, b1, W2, b2 = convs[l]
        w1e = W1[2 * H:]
        b1r = b1.reshape(1, H)
        b2r = b2.reshape(1, 2 * H)
        g_a = _sc_gather_h(hs, hd, src_h[0], dst_h[0])
        g_b = _sc_gather_h(hs, hd, src_h[1], dst_h[1])
        msg_a = _edge_stage(g_a, edge_attr, 0, w1e, b1r, W2, b2r)
        msg_b = _edge_stage(g_b, edge_attr, 1, w1e, b1r, W2, b2r)
        agg_a = _sc_scatter_h(msg_a, dst_h[0], zrows)
        agg_b = _sc_scatter_h(msg_b, dst_h[1], zrows)
        if l < 2:
            w1s = convs[l + 1][0][:H]
            w1d = convs[l + 1][0][H:2 * H]
            h, hs, hd = _node_mid(h, agg_a, agg_b, w1s, w1d)
        else:
            h = _node_fin(h, agg_a, agg_b)

    sums, maxs, cnts = _sc_pool(h, batch)
    return _combine(sums, maxs, cnts, W_mlp1, b_mlp1.reshape(1, H),
                    W_mlp2.reshape(1, H), b_mlp2.reshape(1, 1))
